# Initial kernel scaffold; baseline (speedup 1.0000x reference)
import jax
import jax.numpy as jnp
from jax.experimental import pallas as pl


def kernel(pred0, pred1, pred2, W, b):
    raise NotImplementedError("write your pallas kernel here")


# speedup vs baseline: 9.6551x; 9.6551x over previous
import functools

import jax
import jax.numpy as jnp
from jax import lax
from jax.experimental import pallas as pl
from jax.experimental.pallas import tpu as pltpu
from jax.experimental.pallas import tpu_sc as plsc
from jax._src.pallas import mpmd as _mpmd

L = 2048
SD = 8
MS = 64
DIN = 1024
PD = MS * SD
CW = L * SD
PW = 3 * PD
BM = 256
NWORK = 32
RPW = L // NWORK
WRAP0 = L - (MS - 1)


def _matmul_body(x_ref, w_ref, b_ref, o_ref):
    acc = jnp.dot(x_ref[...], w_ref[...], preferred_element_type=jnp.float32)
    acc = acc + b_ref[...]
    o_ref[:, 0:PD] = jnp.zeros((BM, PD), jnp.float32)
    o_ref[:, PD:2 * PD] = acc
    o_ref[:, 2 * PD:PW] = jnp.zeros((BM, PD), jnp.float32)


def _zero_body(o_ref):
    o_ref[...] = jnp.zeros_like(o_ref)


def _sc_expand_body(p_hbm, z_hbm, out_hbm, buf, sem):
    del z_hbm
    wid = lax.axis_index("s") * 2 + lax.axis_index("c")
    base = wid * RPW

    pltpu.sync_copy(p_hbm.at[pl.ds(base, RPW), :], buf)

    def issue(r, carry):
        l = base + r
        a = jnp.minimum(SD * l, CW - PD)
        off = PD + a - SD * l
        pltpu.make_async_copy(
            buf.at[r, pl.ds(off, PD)],
            out_hbm.at[l, pl.ds(a, PD)],
            sem,
        ).start()

        @pl.when(l >= WRAP0)
        def _():
            off2 = PD + CW - SD * l
            pltpu.make_async_copy(
                buf.at[r, pl.ds(off2, PD)],
                out_hbm.at[l, pl.ds(0, PD)],
                sem,
            ).start()

        return carry

    lax.fori_loop(0, RPW, issue, 0)

    n_wrap = jnp.clip(base + RPW - WRAP0, 0, MS - 1)

    def drain(i, carry):
        pltpu.make_async_copy(
            p_hbm.at[0, pl.ds(0, PD)],
            buf.at[0, pl.ds(0, PD)],
            sem,
        ).wait()
        return carry

    lax.fori_loop(0, RPW + n_wrap, drain, 0)


def kernel(pred0, pred1, pred2, W, b):
    del pred0, pred2
    x = pred1.reshape(L, DIN)
    b2 = b.reshape(1, PD)

    p_pad = pl.pallas_call(
        _matmul_body,
        grid=(L // BM,),
        in_specs=[
            pl.BlockSpec((BM, DIN), lambda i: (i, 0)),
            pl.BlockSpec((DIN, PD), lambda i: (0, 0)),
            pl.BlockSpec((1, PD), lambda i: (0, 0)),
        ],
        out_specs=pl.BlockSpec((BM, PW), lambda i: (i, 0)),
        out_shape=jax.ShapeDtypeStruct((L, PW), jnp.float32),
    )(x, W, b2)

    zeros_buf = pl.pallas_call(
        _zero_body,
        grid=(16,),
        out_specs=pl.BlockSpec((L // 16, CW), lambda i: (i, 0)),
        out_shape=jax.ShapeDtypeStruct((L, CW), jnp.float32),
    )()

    mesh = plsc.VectorSubcoreMesh(core_axis_name="c", subcore_axis_name="s")
    expand = _mpmd._mpmd_map(
        [(mesh, _sc_expand_body)],
        out_types=jax.ShapeDtypeStruct((L, CW), jnp.float32),
        input_output_aliases={1: 0},
        compiler_params=pltpu.CompilerParams(use_tc_tiling_on_sc=False),
        scratch_types=[
            pltpu.VMEM((RPW, PW), jnp.float32),
            pltpu.SemaphoreType.DMA,
        ],
    )
    out = expand(p_pad, zeros_buf)
    return out.reshape(1, L, L, SD)
